# baseline (speedup 1.0000x reference)
import jax
import jax.numpy as jnp
from jax.experimental import pallas as pl
from jax.experimental.pallas import tpu as pltpu

_CHUNKS = 8


def _wide_kernel(w_ref, x_hbm, b_ref, o_ref, x_vmem, sem_in):
    kdim, bdim = x_hbm.shape
    cols = bdim // _CHUNKS
    cps = [
        pltpu.make_async_copy(
            x_hbm.at[:, pl.ds(i * cols, cols)],
            x_vmem.at[:, pl.ds(i * cols, cols)],
            sem_in.at[i],
        )
        for i in range(_CHUNKS)
    ]
    for cp in cps:
        cp.start()
    w = w_ref[...]
    b = b_ref[...][:, None]
    for i in range(_CHUNKS):
        cps[i].wait()
        sl = pl.ds(i * cols, cols)
        o_ref[:, sl] = (
            jnp.dot(w, x_vmem[:, sl], preferred_element_type=jnp.float32) + b
        )


def kernel(feat, server_model, len_seq, mask, manfeat, emb1_w, emb2_w, emb3_w,
           emb4_w, emb5_w, k_w, o_w, cls_w, cls_b, wide_w, wide_b):
    b, k = manfeat.shape
    n = wide_w.shape[1]
    xt = manfeat.T
    xt = pltpu.with_memory_space_constraint(xt, pltpu.MemorySpace.HBM)
    wt = wide_w.T
    out_t = pl.pallas_call(
        _wide_kernel,
        in_specs=[
            pl.BlockSpec(memory_space=pltpu.MemorySpace.VMEM),
            pl.BlockSpec(memory_space=pltpu.MemorySpace.HBM),
            pl.BlockSpec(memory_space=pltpu.MemorySpace.VMEM),
        ],
        out_specs=pl.BlockSpec(memory_space=pltpu.MemorySpace.VMEM),
        out_shape=jax.ShapeDtypeStruct((n, b), jnp.float32),
        scratch_shapes=[
            pltpu.VMEM((k, b), jnp.float32),
            pltpu.SemaphoreType.DMA((_CHUNKS,)),
        ],
    )(wt, xt, wide_b)
    return out_t.T


# speedup vs baseline: 1.0637x; 1.0637x over previous
import jax
import jax.numpy as jnp
from jax.experimental import pallas as pl
from jax.experimental.pallas import tpu as pltpu

_CHUNKS = 5


def _wide_kernel(w_ref, x_hbm, b_ref, o_ref, x_vmem, sem_in):
    kdim, bdim = x_hbm.shape
    kc = kdim // _CHUNKS
    cps = [
        pltpu.make_async_copy(
            x_hbm.at[pl.ds(i * kc, kc), :],
            x_vmem.at[pl.ds(i * kc, kc), :],
            sem_in.at[i],
        )
        for i in range(_CHUNKS)
    ]
    for cp in cps:
        cp.start()
    acc = b_ref[...][:, None]
    for i in range(_CHUNKS):
        cps[i].wait()
        sl = pl.ds(i * kc, kc)
        acc = acc + jnp.dot(
            w_ref[:, sl], x_vmem[sl, :], preferred_element_type=jnp.float32
        )
    o_ref[...] = acc


def kernel(feat, server_model, len_seq, mask, manfeat, emb1_w, emb2_w, emb3_w,
           emb4_w, emb5_w, k_w, o_w, cls_w, cls_b, wide_w, wide_b):
    b, k = manfeat.shape
    n = wide_w.shape[1]
    xt = manfeat.T
    xt = pltpu.with_memory_space_constraint(xt, pltpu.MemorySpace.HBM)
    wt = wide_w.T
    out_t = pl.pallas_call(
        _wide_kernel,
        in_specs=[
            pl.BlockSpec(memory_space=pltpu.MemorySpace.VMEM),
            pl.BlockSpec(memory_space=pltpu.MemorySpace.HBM),
            pl.BlockSpec(memory_space=pltpu.MemorySpace.VMEM),
        ],
        out_specs=pl.BlockSpec(memory_space=pltpu.MemorySpace.VMEM),
        out_shape=jax.ShapeDtypeStruct((n, b), jnp.float32),
        scratch_shapes=[
            pltpu.VMEM((k, b), jnp.float32),
            pltpu.SemaphoreType.DMA((_CHUNKS,)),
        ],
    )(wt, xt, wide_b)
    return out_t.T


# speedup vs baseline: 1.3229x; 1.2436x over previous
import jax
import jax.numpy as jnp
from jax.experimental import pallas as pl
from jax.experimental.pallas import tpu as pltpu


def _wide_kernel(w_ref, x_ref, b_ref, o_ref):
    o_ref[...] = (
        jnp.dot(w_ref[...], x_ref[...], preferred_element_type=jnp.float32)
        + b_ref[...][:, None]
    )


def kernel(feat, server_model, len_seq, mask, manfeat, emb1_w, emb2_w, emb3_w,
           emb4_w, emb5_w, k_w, o_w, cls_w, cls_b, wide_w, wide_b):
    b, k = manfeat.shape
    n = wide_w.shape[1]
    xt = manfeat.T
    wt = wide_w.T
    out_t = pl.pallas_call(
        _wide_kernel,
        out_shape=jax.ShapeDtypeStruct((n, b), jnp.float32),
        compiler_params=pltpu.CompilerParams(
            skip_device_barrier=True,
            disable_bounds_checks=True,
            disable_semaphore_checks=True,
        ),
    )(wt, xt, wide_b)
    return out_t.T
